# baseline (speedup 1.0000x reference)
import functools

import jax
import jax.numpy as jnp
from jax import lax
from jax.experimental import pallas as pl
from jax.experimental.pallas import tpu as pltpu
from jax.experimental.pallas import tpu_sc as plsc

_N = 10000
_F = 128
_E = 320000
_NPAD = 10240
_CHUNK = 128
_NTILES = 32
_CPT = 80
_EPAD = _NTILES * _CPT * _CHUNK
_RPT = _NPAD // 16
_BLK = 1024
_NBLK = -(-_N // _BLK)


def _gru_body(xt_ref, wih_ref, whh_ref, bih_ref, bhh_ref, w0_ref, w_ref):
    xt = xt_ref[...]
    w0 = w0_ref[...]
    gi = lax.dot_general(xt, wih_ref[...], (((1,), (1,)), ((), ())),
                         preferred_element_type=jnp.float32) + bih_ref[...][None, :]
    gh = lax.dot_general(w0, whh_ref[...], (((1,), (1,)), ((), ())),
                         preferred_element_type=jnp.float32) + bhh_ref[...][None, :]
    r = jax.nn.sigmoid(gi[:, 0:_F] + gh[:, 0:_F])
    z = jax.nn.sigmoid(gi[:, _F:2 * _F] + gh[:, _F:2 * _F])
    cand = jnp.tanh(gi[:, 2 * _F:3 * _F] + r * gh[:, 2 * _F:3 * _F])
    w_ref[...] = (1.0 - z) * cand + z * w0


_gru_call = pl.pallas_call(
    _gru_body,
    out_shape=jax.ShapeDtypeStruct((_F, _F), jnp.float32),
)


def _prep_body(x_ref, w_ref, dp_ref, xs_ref, xd_ref, dinv_ref):
    deg = dp_ref[0, :] + dp_ref[1, :] + 1.0
    dinv = jnp.where(deg > 0, lax.rsqrt(jnp.where(deg > 0, deg, 1.0)), 0.0)
    xw = jnp.dot(x_ref[...], w_ref[...], preferred_element_type=jnp.float32)
    xs = xw * dinv[:, None]
    xs_ref[...] = xs
    xd_ref[...] = xs * dinv[:, None]
    dinv_ref[...] = dinv


_prep_call = pl.pallas_call(
    _prep_body,
    grid=(_NBLK,),
    in_specs=[
        pl.BlockSpec((_BLK, _F), lambda i: (i, 0)),
        pl.BlockSpec((_F, _F), lambda i: (0, 0)),
        pl.BlockSpec((2, _BLK), lambda i: (0, i)),
    ],
    out_specs=[
        pl.BlockSpec((_BLK, _F), lambda i: (i, 0)),
        pl.BlockSpec((_BLK, _F), lambda i: (i, 0)),
        pl.BlockSpec((_BLK,), lambda i: (i,)),
    ],
    out_shape=[
        jax.ShapeDtypeStruct((_N, _F), jnp.float32),
        jax.ShapeDtypeStruct((_N, _F), jnp.float32),
        jax.ShapeDtypeStruct((_N,), jnp.float32),
    ],
)


def _head_body(acc_ref, xd_ref, dinv_ref, wl_ref, bl_ref, out_ref):
    m = (acc_ref[0] + acc_ref[1]) * dinv_ref[...][:, None] + xd_ref[...]
    h = jnp.maximum(m, 0.0)
    y = jnp.sum(h * wl_ref[...], axis=1, keepdims=True)
    out_ref[...] = y + bl_ref[0]


_head_call = pl.pallas_call(
    _head_body,
    grid=(_NBLK,),
    in_specs=[
        pl.BlockSpec((2, _BLK, _F), lambda i: (0, i, 0)),
        pl.BlockSpec((_BLK, _F), lambda i: (i, 0)),
        pl.BlockSpec((_BLK,), lambda i: (i,)),
        pl.BlockSpec((1, _F), lambda i: (0, 0)),
        pl.BlockSpec(memory_space=pltpu.SMEM),
    ],
    out_specs=pl.BlockSpec((_BLK, 1), lambda i: (i, 0)),
    out_shape=jax.ShapeDtypeStruct((_N, 1), jnp.float32),
)


def _deg_body(dd_ref, ew_ref, out_ref, dd_all, ew_all, stage_v, sem, acc_sh):
    cid = lax.axis_index("c")
    sid = lax.axis_index("s")
    wid = cid * 16 + sid
    for k in range(_RPT // 16):
        stage_v[pl.ds(k * 16, 16)] = jnp.zeros((16,), jnp.float32)
    pltpu.sync_copy(stage_v, acc_sh.at[pl.ds(sid * _RPT, _RPT)])
    plsc.subcore_barrier()
    pltpu.sync_copy(dd_ref.at[pl.ds(wid * _CPT, _CPT)], dd_all)
    pltpu.sync_copy(ew_ref.at[pl.ds(wid * _CPT, _CPT)], ew_all)

    def wave(i, carry):
        for b in range(8):
            j = i * 8 + b
            pltpu.async_copy(ew_all.at[j], acc_sh.at[dd_all.at[j]], sem,
                             add=True)
        for b in range(8):
            j = i * 8 + b
            pltpu.make_async_copy(ew_all.at[j], acc_sh.at[dd_all.at[j]],
                                  sem).wait()
        return carry

    lax.fori_loop(0, _CPT // 8, wave, 0)
    plsc.subcore_barrier()
    pltpu.sync_copy(acc_sh.at[pl.ds(sid * _RPT, _RPT)], stage_v)
    pltpu.sync_copy(stage_v, out_ref.at[cid, pl.ds(sid * _RPT, _RPT)])


_deg_call = pl.kernel(
    _deg_body,
    out_type=jax.ShapeDtypeStruct((2, _NPAD), jnp.float32),
    mesh=plsc.VectorSubcoreMesh(core_axis_name="c", subcore_axis_name="s"),
    scratch_types=[
        pltpu.VMEM((_CPT, _CHUNK), jnp.int32),
        pltpu.VMEM((_CPT, _CHUNK), jnp.float32),
        pltpu.VMEM((_RPT,), jnp.float32),
        pltpu.SemaphoreType.DMA,
        pltpu.VMEM_SHARED((_NPAD,), jnp.float32),
    ],
    compiler_params=pltpu.CompilerParams(needs_layout_passes=False),
)


def _msg_body(pk_ref, ew_ref, xs_ref, zz_ref, out_ref,
              pk0, pk1, ew0, ew1, rows0, rows1, sem0, sem1, ssem0, ssem1,
              acc_sh):
    cid = lax.axis_index("c")
    sid = lax.axis_index("s")
    wid = cid * 16 + sid
    base = wid * _CPT
    bufs = ((pk0, ew0, rows0, sem0, ssem0), (pk1, ew1, rows1, sem1, ssem1))

    def stage_and_fire(c, pk_b, ew_b, rows_b, sem_b, ssem_b):
        pltpu.make_async_copy(rows_b, acc_sh.at[pk_b.at[1]], ssem_b).wait()
        pltpu.sync_copy(pk_ref.at[c], pk_b)
        pltpu.sync_copy(ew_ref.at[c], ew_b)
        pltpu.async_copy(xs_ref.at[pk_b.at[0]], rows_b, sem_b)

    pltpu.sync_copy(zz_ref, rows0)
    pltpu.sync_copy(zz_ref, rows1)
    for k in range(_RPT // _CHUNK):
        pltpu.sync_copy(rows0, acc_sh.at[pl.ds(sid * _RPT + k * _CHUNK, _CHUNK)])
    plsc.subcore_barrier()

    pltpu.sync_copy(pk_ref.at[base], pk0)
    pltpu.async_copy(rows0, acc_sh.at[pk0.at[1]], ssem0, add=True)
    pltpu.async_copy(rows1, acc_sh.at[pk0.at[1]], ssem1, add=True)
    stage_and_fire(base, *bufs[0])

    def pair(i, carry):
        for b in range(2):
            j = 2 * i + b
            pk_b, ew_b, rows_b, sem_b, ssem_b = bufs[b]
            stage_and_fire(base + j + 1, *bufs[1 - b])
            pltpu.make_async_copy(xs_ref.at[pk_b.at[0]], rows_b, sem_b).wait()
            def scale(g, carry2, ew_b=ew_b, rows_b=rows_b):
                for r in range(8):
                    e = g * 8 + r
                    bc = plsc.load_gather(
                        ew_b, [jnp.full((16,), e, dtype=jnp.int32)])
                    for fb in range(_F // 16):
                        sl = pl.ds(fb * 16, 16)
                        rows_b[e, sl] = rows_b[e, sl] * bc
                return carry2
            lax.fori_loop(0, _CHUNK // 8, scale, 0)
            pltpu.async_copy(rows_b, acc_sh.at[pk_b.at[1]], ssem_b, add=True)
        return carry

    lax.fori_loop(0, _CPT // 2, pair, 0)
    pltpu.make_async_copy(xs_ref.at[pk0.at[0]], rows0, sem0).wait()
    pltpu.make_async_copy(rows1, acc_sh.at[pk1.at[1]], ssem1).wait()
    plsc.subcore_barrier()
    for k in range(_RPT // _CHUNK):
        off = sid * _RPT + k * _CHUNK
        pltpu.sync_copy(acc_sh.at[pl.ds(off, _CHUNK)], rows0)
        pltpu.sync_copy(rows0, out_ref.at[cid, pl.ds(off, _CHUNK)])


_msg_call = pl.kernel(
    _msg_body,
    out_type=jax.ShapeDtypeStruct((2, _NPAD, _F), jnp.float32),
    mesh=plsc.VectorSubcoreMesh(core_axis_name="c", subcore_axis_name="s"),
    scratch_types=[
        pltpu.VMEM((2, _CHUNK), jnp.int32),
        pltpu.VMEM((2, _CHUNK), jnp.int32),
        pltpu.VMEM((_CHUNK,), jnp.float32),
        pltpu.VMEM((_CHUNK,), jnp.float32),
        pltpu.VMEM((_CHUNK, _F), jnp.float32),
        pltpu.VMEM((_CHUNK, _F), jnp.float32),
        pltpu.SemaphoreType.DMA,
        pltpu.SemaphoreType.DMA,
        pltpu.SemaphoreType.DMA,
        pltpu.SemaphoreType.DMA,
        pltpu.VMEM_SHARED((_NPAD, _F), jnp.float32),
    ],
    compiler_params=pltpu.CompilerParams(needs_layout_passes=False),
)


def kernel(x, edge_index, edge_weight, p, W_ih, W_hh, b_ih, b_hh, W0, W_lin, b_lin):
    score = (x @ p) / jnp.linalg.norm(p)
    topv, perm = lax.top_k(score, _F)
    x_tilde = x[perm] * jnp.tanh(topv)[:, None]

    W = _gru_call(x_tilde, W_ih, W_hh, b_ih, b_hh, W0)

    pad = _EPAD - _E + _CHUNK
    ncht = _NTILES * _CPT + 1
    src2 = jnp.concatenate(
        [edge_index[0], jnp.zeros((pad,), jnp.int32)]).reshape(ncht, 1, _CHUNK)
    dst2 = jnp.concatenate(
        [edge_index[1], jnp.zeros((pad,), jnp.int32)]).reshape(ncht, 1, _CHUNK)
    pk = jnp.concatenate([src2, dst2], axis=1)
    dd = dst2.reshape(ncht, _CHUNK)
    ew2 = jnp.concatenate(
        [edge_weight, jnp.zeros((pad,), jnp.float32)]).reshape(ncht, _CHUNK)

    degp = _deg_call(dd, ew2)
    xs, xd, dinv = _prep_call(x, W, degp)

    zz = jnp.zeros((_CHUNK, _F), jnp.float32)
    acc = _msg_call(pk, ew2, xs, zz)

    return _head_call(acc, xd, dinv, W_lin, b_lin)


# speedup vs baseline: 1.0070x; 1.0070x over previous
import functools

import jax
import jax.numpy as jnp
from jax import lax
from jax.experimental import pallas as pl
from jax.experimental.pallas import tpu as pltpu
from jax.experimental.pallas import tpu_sc as plsc

_N = 10000
_F = 128
_E = 320000
_NPAD = 10240
_CHUNK = 128
_NTILES = 32
_CPT = 80
_EPAD = _NTILES * _CPT * _CHUNK
_RPT = _NPAD // 16
_BLK = 1024
_NBLK = -(-_N // _BLK)


def _gru_body(xt_ref, wih_ref, whh_ref, bih_ref, bhh_ref, w0_ref, w_ref):
    xt = xt_ref[...]
    w0 = w0_ref[...]
    gi = lax.dot_general(xt, wih_ref[...], (((1,), (1,)), ((), ())),
                         preferred_element_type=jnp.float32) + bih_ref[...][None, :]
    gh = lax.dot_general(w0, whh_ref[...], (((1,), (1,)), ((), ())),
                         preferred_element_type=jnp.float32) + bhh_ref[...][None, :]
    r = jax.nn.sigmoid(gi[:, 0:_F] + gh[:, 0:_F])
    z = jax.nn.sigmoid(gi[:, _F:2 * _F] + gh[:, _F:2 * _F])
    cand = jnp.tanh(gi[:, 2 * _F:3 * _F] + r * gh[:, 2 * _F:3 * _F])
    w_ref[...] = (1.0 - z) * cand + z * w0


_gru_call = pl.pallas_call(
    _gru_body,
    out_shape=jax.ShapeDtypeStruct((_F, _F), jnp.float32),
)


def _prep_body(x_ref, w_ref, dp_ref, xs_ref, xd_ref, dinv_ref):
    deg = dp_ref[0, :] + dp_ref[1, :] + 1.0
    dinv = jnp.where(deg > 0, lax.rsqrt(jnp.where(deg > 0, deg, 1.0)), 0.0)
    xw = jnp.dot(x_ref[...], w_ref[...], preferred_element_type=jnp.float32)
    xs = xw * dinv[:, None]
    xs_ref[...] = xs
    xd_ref[...] = xs * dinv[:, None]
    dinv_ref[...] = dinv


_prep_call = pl.pallas_call(
    _prep_body,
    grid=(_NBLK,),
    in_specs=[
        pl.BlockSpec((_BLK, _F), lambda i: (i, 0)),
        pl.BlockSpec((_F, _F), lambda i: (0, 0)),
        pl.BlockSpec((2, _BLK), lambda i: (0, i)),
    ],
    out_specs=[
        pl.BlockSpec((_BLK, _F), lambda i: (i, 0)),
        pl.BlockSpec((_BLK, _F), lambda i: (i, 0)),
        pl.BlockSpec((_BLK,), lambda i: (i,)),
    ],
    out_shape=[
        jax.ShapeDtypeStruct((_N, _F), jnp.float32),
        jax.ShapeDtypeStruct((_N, _F), jnp.float32),
        jax.ShapeDtypeStruct((_N,), jnp.float32),
    ],
)


def _head_body(acc_ref, xd_ref, dinv_ref, wl_ref, bl_ref, out_ref):
    m = (acc_ref[0] + acc_ref[1]) * dinv_ref[...][:, None] + xd_ref[...]
    h = jnp.maximum(m, 0.0)
    y = jnp.sum(h * wl_ref[...], axis=1, keepdims=True)
    out_ref[...] = y + bl_ref[0]


_head_call = pl.pallas_call(
    _head_body,
    grid=(_NBLK,),
    in_specs=[
        pl.BlockSpec((2, _BLK, _F), lambda i: (0, i, 0)),
        pl.BlockSpec((_BLK, _F), lambda i: (i, 0)),
        pl.BlockSpec((_BLK,), lambda i: (i,)),
        pl.BlockSpec((1, _F), lambda i: (0, 0)),
        pl.BlockSpec(memory_space=pltpu.SMEM),
    ],
    out_specs=pl.BlockSpec((_BLK, 1), lambda i: (i, 0)),
    out_shape=jax.ShapeDtypeStruct((_N, 1), jnp.float32),
)


def _deg_body(dd_ref, ew_ref, out_ref, dd_all, ew_all, stage_v, sem, acc_sh):
    cid = lax.axis_index("c")
    sid = lax.axis_index("s")
    wid = cid * 16 + sid
    for k in range(_RPT // 16):
        stage_v[pl.ds(k * 16, 16)] = jnp.zeros((16,), jnp.float32)
    pltpu.sync_copy(stage_v, acc_sh.at[pl.ds(sid * _RPT, _RPT)])
    plsc.subcore_barrier()
    pltpu.sync_copy(dd_ref.at[pl.ds(wid * _CPT, _CPT)], dd_all)
    pltpu.sync_copy(ew_ref.at[pl.ds(wid * _CPT, _CPT)], ew_all)

    def wave(i, carry):
        for b in range(8):
            j = i * 8 + b
            pltpu.async_copy(ew_all.at[j], acc_sh.at[dd_all.at[j]], sem,
                             add=True)
        for b in range(8):
            j = i * 8 + b
            pltpu.make_async_copy(ew_all.at[j], acc_sh.at[dd_all.at[j]],
                                  sem).wait()
        return carry

    lax.fori_loop(0, _CPT // 8, wave, 0)
    plsc.subcore_barrier()
    pltpu.sync_copy(acc_sh.at[pl.ds(sid * _RPT, _RPT)], stage_v)
    pltpu.sync_copy(stage_v, out_ref.at[cid, pl.ds(sid * _RPT, _RPT)])


_deg_call = pl.kernel(
    _deg_body,
    out_type=jax.ShapeDtypeStruct((2, _NPAD), jnp.float32),
    mesh=plsc.VectorSubcoreMesh(core_axis_name="c", subcore_axis_name="s"),
    scratch_types=[
        pltpu.VMEM((_CPT, _CHUNK), jnp.int32),
        pltpu.VMEM((_CPT, _CHUNK), jnp.float32),
        pltpu.VMEM((_RPT,), jnp.float32),
        pltpu.SemaphoreType.DMA,
        pltpu.VMEM_SHARED((_NPAD,), jnp.float32),
    ],
    compiler_params=pltpu.CompilerParams(needs_layout_passes=False),
)


def _msg_body(pk_ref, ew_ref, xs_ref, zz_ref, out_ref,
              pk0, pk1, ew0, ew1, rows0, rows1, sem0, sem1, ssem0, ssem1,
              acc_sh):
    cid = lax.axis_index("c")
    sid = lax.axis_index("s")
    wid = cid * 16 + sid
    base = wid * _CPT
    bufs = ((pk0, ew0, rows0, sem0, ssem0), (pk1, ew1, rows1, sem1, ssem1))

    def stage_and_fire(c, pk_b, ew_b, rows_b, sem_b, ssem_b):
        pltpu.sync_copy(pk_ref.at[c], pk_b)
        pltpu.sync_copy(ew_ref.at[c], ew_b)
        pltpu.async_copy(xs_ref.at[pk_b.at[0]], rows_b, sem_b)

    pltpu.sync_copy(zz_ref, rows0)
    for k in range(_RPT // _CHUNK):
        pltpu.sync_copy(rows0, acc_sh.at[pl.ds(sid * _RPT + k * _CHUNK, _CHUNK)])
    plsc.subcore_barrier()

    stage_and_fire(base, *bufs[0])

    def pair(i, carry):
        for b in range(2):
            j = 2 * i + b
            pk_b, ew_b, rows_b, sem_b, ssem_b = bufs[b]
            stage_and_fire(base + j + 1, *bufs[1 - b])
            pltpu.make_async_copy(xs_ref.at[pk_b.at[0]], rows_b, sem_b).wait()
            def scale(g, carry2, ew_b=ew_b, rows_b=rows_b):
                for r in range(8):
                    e = g * 8 + r
                    bc = plsc.load_gather(
                        ew_b, [jnp.full((16,), e, dtype=jnp.int32)])
                    for fb in range(_F // 16):
                        sl = pl.ds(fb * 16, 16)
                        rows_b[e, sl] = rows_b[e, sl] * bc
                return carry2
            lax.fori_loop(0, _CHUNK // 8, scale, 0)
            pltpu.sync_copy(rows_b, acc_sh.at[pk_b.at[1]], add=True)
        return carry

    lax.fori_loop(0, _CPT // 2, pair, 0)
    pltpu.make_async_copy(xs_ref.at[pk0.at[0]], rows0, sem0).wait()
    plsc.subcore_barrier()
    for k in range(_RPT // _CHUNK):
        off = sid * _RPT + k * _CHUNK
        pltpu.sync_copy(acc_sh.at[pl.ds(off, _CHUNK)], rows0)
        pltpu.sync_copy(rows0, out_ref.at[cid, pl.ds(off, _CHUNK)])


_msg_call = pl.kernel(
    _msg_body,
    out_type=jax.ShapeDtypeStruct((2, _NPAD, _F), jnp.float32),
    mesh=plsc.VectorSubcoreMesh(core_axis_name="c", subcore_axis_name="s"),
    scratch_types=[
        pltpu.VMEM((2, _CHUNK), jnp.int32),
        pltpu.VMEM((2, _CHUNK), jnp.int32),
        pltpu.VMEM((_CHUNK,), jnp.float32),
        pltpu.VMEM((_CHUNK,), jnp.float32),
        pltpu.VMEM((_CHUNK, _F), jnp.float32),
        pltpu.VMEM((_CHUNK, _F), jnp.float32),
        pltpu.SemaphoreType.DMA,
        pltpu.SemaphoreType.DMA,
        pltpu.SemaphoreType.DMA,
        pltpu.SemaphoreType.DMA,
        pltpu.VMEM_SHARED((_NPAD, _F), jnp.float32),
    ],
    compiler_params=pltpu.CompilerParams(needs_layout_passes=False),
)


def kernel(x, edge_index, edge_weight, p, W_ih, W_hh, b_ih, b_hh, W0, W_lin, b_lin):
    score = (x @ p) / jnp.linalg.norm(p)
    topv, perm = lax.top_k(score, _F)
    x_tilde = x[perm] * jnp.tanh(topv)[:, None]

    W = _gru_call(x_tilde, W_ih, W_hh, b_ih, b_hh, W0)

    pad = _EPAD - _E + _CHUNK
    ncht = _NTILES * _CPT + 1
    src2 = jnp.concatenate(
        [edge_index[0], jnp.zeros((pad,), jnp.int32)]).reshape(ncht, 1, _CHUNK)
    dst2 = jnp.concatenate(
        [edge_index[1], jnp.zeros((pad,), jnp.int32)]).reshape(ncht, 1, _CHUNK)
    pk = jnp.concatenate([src2, dst2], axis=1)
    dd = dst2.reshape(ncht, _CHUNK)
    ew2 = jnp.concatenate(
        [edge_weight, jnp.zeros((pad,), jnp.float32)]).reshape(ncht, _CHUNK)

    degp = _deg_call(dd, ew2)
    xs, xd, dinv = _prep_call(x, W, degp)

    zz = jnp.zeros((_CHUNK, _F), jnp.float32)
    acc = _msg_call(pk, ew2, xs, zz)

    return _head_call(acc, xd, dinv, W_lin, b_lin)


# speedup vs baseline: 1.1040x; 1.0964x over previous
import functools

import jax
import jax.numpy as jnp
from jax import lax
from jax.experimental import pallas as pl
from jax.experimental.pallas import tpu as pltpu
from jax.experimental.pallas import tpu_sc as plsc

_N = 10000
_F = 128
_E = 320000
_NPAD = 10240
_CHUNK = 128
_NTILES = 32
_CPT = 80
_EPAD = _NTILES * _CPT * _CHUNK
_RPT = _NPAD // 16
_BLK = 1024
_NBLK = -(-_N // _BLK)


def _gru_body(xt_ref, wih_ref, whh_ref, bih_ref, bhh_ref, w0_ref, w_ref):
    xt = xt_ref[...]
    w0 = w0_ref[...]
    gi = lax.dot_general(xt, wih_ref[...], (((1,), (1,)), ((), ())),
                         preferred_element_type=jnp.float32) + bih_ref[...][None, :]
    gh = lax.dot_general(w0, whh_ref[...], (((1,), (1,)), ((), ())),
                         preferred_element_type=jnp.float32) + bhh_ref[...][None, :]
    r = jax.nn.sigmoid(gi[:, 0:_F] + gh[:, 0:_F])
    z = jax.nn.sigmoid(gi[:, _F:2 * _F] + gh[:, _F:2 * _F])
    cand = jnp.tanh(gi[:, 2 * _F:3 * _F] + r * gh[:, 2 * _F:3 * _F])
    w_ref[...] = (1.0 - z) * cand + z * w0


_gru_call = pl.pallas_call(
    _gru_body,
    out_shape=jax.ShapeDtypeStruct((_F, _F), jnp.float32),
)


def _prep_body(x_ref, w_ref, dp_ref, xs_ref, xd_ref, dinv_ref):
    deg = dp_ref[0, :] + dp_ref[1, :] + 1.0
    dinv = jnp.where(deg > 0, lax.rsqrt(jnp.where(deg > 0, deg, 1.0)), 0.0)
    xw = jnp.dot(x_ref[...], w_ref[...], preferred_element_type=jnp.float32)
    xs = xw * dinv[:, None]
    xs_ref[...] = xs
    xd_ref[...] = xs * dinv[:, None]
    dinv_ref[...] = dinv


_prep_call = pl.pallas_call(
    _prep_body,
    grid=(_NBLK,),
    in_specs=[
        pl.BlockSpec((_BLK, _F), lambda i: (i, 0)),
        pl.BlockSpec((_F, _F), lambda i: (0, 0)),
        pl.BlockSpec((2, _BLK), lambda i: (0, i)),
    ],
    out_specs=[
        pl.BlockSpec((_BLK, _F), lambda i: (i, 0)),
        pl.BlockSpec((_BLK, _F), lambda i: (i, 0)),
        pl.BlockSpec((_BLK,), lambda i: (i,)),
    ],
    out_shape=[
        jax.ShapeDtypeStruct((_N, _F), jnp.float32),
        jax.ShapeDtypeStruct((_N, _F), jnp.float32),
        jax.ShapeDtypeStruct((_N,), jnp.float32),
    ],
)


def _head_body(acc_ref, xd_ref, dinv_ref, wl_ref, bl_ref, out_ref):
    m = (acc_ref[0] + acc_ref[1]) * dinv_ref[...][:, None] + xd_ref[...]
    h = jnp.maximum(m, 0.0)
    y = jnp.sum(h * wl_ref[...], axis=1, keepdims=True)
    out_ref[...] = y + bl_ref[0]


_head_call = pl.pallas_call(
    _head_body,
    grid=(_NBLK,),
    in_specs=[
        pl.BlockSpec((2, _BLK, _F), lambda i: (0, i, 0)),
        pl.BlockSpec((_BLK, _F), lambda i: (i, 0)),
        pl.BlockSpec((_BLK,), lambda i: (i,)),
        pl.BlockSpec((1, _F), lambda i: (0, 0)),
        pl.BlockSpec(memory_space=pltpu.SMEM),
    ],
    out_specs=pl.BlockSpec((_BLK, 1), lambda i: (i, 0)),
    out_shape=jax.ShapeDtypeStruct((_N, 1), jnp.float32),
)


def _deg_body(dd_ref, ew_ref, out_ref, dd_all, ew_all, stage_v, sem, acc_sh):
    cid = lax.axis_index("c")
    sid = lax.axis_index("s")
    wid = cid * 16 + sid
    for k in range(_RPT // 16):
        stage_v[pl.ds(k * 16, 16)] = jnp.zeros((16,), jnp.float32)
    pltpu.sync_copy(stage_v, acc_sh.at[pl.ds(sid * _RPT, _RPT)])
    plsc.subcore_barrier()
    pltpu.sync_copy(dd_ref.at[pl.ds(wid * _CPT, _CPT)], dd_all)
    pltpu.sync_copy(ew_ref.at[pl.ds(wid * _CPT, _CPT)], ew_all)

    def wave(i, carry):
        for b in range(8):
            j = i * 8 + b
            pltpu.async_copy(ew_all.at[j], acc_sh.at[dd_all.at[j]], sem,
                             add=True)
        for b in range(8):
            j = i * 8 + b
            pltpu.make_async_copy(ew_all.at[j], acc_sh.at[dd_all.at[j]],
                                  sem).wait()
        return carry

    lax.fori_loop(0, _CPT // 8, wave, 0)
    plsc.subcore_barrier()
    pltpu.sync_copy(acc_sh.at[pl.ds(sid * _RPT, _RPT)], stage_v)
    pltpu.sync_copy(stage_v, out_ref.at[cid, pl.ds(sid * _RPT, _RPT)])


_deg_call = pl.kernel(
    _deg_body,
    out_type=jax.ShapeDtypeStruct((2, _NPAD), jnp.float32),
    mesh=plsc.VectorSubcoreMesh(core_axis_name="c", subcore_axis_name="s"),
    scratch_types=[
        pltpu.VMEM((_CPT, _CHUNK), jnp.int32),
        pltpu.VMEM((_CPT, _CHUNK), jnp.float32),
        pltpu.VMEM((_RPT,), jnp.float32),
        pltpu.SemaphoreType.DMA,
        pltpu.VMEM_SHARED((_NPAD,), jnp.float32),
    ],
    compiler_params=pltpu.CompilerParams(needs_layout_passes=False),
)


_NSTREAM = 4
_SUB = _CHUNK // _NSTREAM


def _msg_body(ss_ref, dd_ref, ew_ref, xs_ref, out_ref,
              src_all, dst0, dst1, ewc0, ewc1, rows0, rows1, sem0, sem1,
              acc_sh):
    cid = lax.axis_index("c")
    sid = lax.axis_index("s")
    wid = cid * 16 + sid
    base = wid * _CPT
    bufs = ((dst0, ewc0, rows0, sem0), (dst1, ewc1, rows1, sem1))

    zv = jnp.zeros((16,), jnp.float32)
    for e in range(_CHUNK):
        for fb in range(_F // 16):
            rows0[e, pl.ds(fb * 16, 16)] = zv
    for k in range(_RPT // _CHUNK):
        pltpu.sync_copy(rows0, acc_sh.at[pl.ds(sid * _RPT + k * _CHUNK, _CHUNK)])
    plsc.subcore_barrier()

    pltpu.sync_copy(ss_ref.at[pl.ds(base, _CPT)], src_all)

    def refill(j, dst_b, ew_b, rows_b, sem_b):
        pltpu.async_copy(dd_ref.at[base + j], dst_b, sem_b)
        pltpu.async_copy(ew_ref.at[base + j], ew_b, sem_b)
        for q in range(_NSTREAM):
            pltpu.async_copy(
                xs_ref.at[src_all.at[j, pl.ds(q * _SUB, _SUB)]],
                rows_b.at[pl.ds(q * _SUB, _SUB)], sem_b)

    def drain(j, dst_b, ew_b, rows_b, sem_b):
        pltpu.make_async_copy(dd_ref.at[base + j], dst_b, sem_b).wait()
        pltpu.make_async_copy(ew_ref.at[base + j], ew_b, sem_b).wait()
        for q in range(_NSTREAM):
            pltpu.make_async_copy(
                xs_ref.at[src_all.at[j, pl.ds(q * _SUB, _SUB)]],
                rows_b.at[pl.ds(q * _SUB, _SUB)], sem_b).wait()

    refill(0, *bufs[0])
    refill(1, *bufs[1])

    def pair(i, carry):
        for b in range(2):
            j = 2 * i + b
            dst_b, ew_b, rows_b, sem_b = bufs[b]
            drain(j, dst_b, ew_b, rows_b, sem_b)

            def scale(g, carry2, ew_b=ew_b, rows_b=rows_b):
                for r in range(8):
                    e = g * 8 + r
                    bc = plsc.load_gather(
                        ew_b, [jnp.full((16,), e, dtype=jnp.int32)])
                    for fb in range(_F // 16):
                        sl = pl.ds(fb * 16, 16)
                        rows_b[e, sl] = rows_b[e, sl] * bc
                return carry2
            lax.fori_loop(0, _CHUNK // 8, scale, 0)
            pltpu.sync_copy(rows_b, acc_sh.at[dst_b], add=True)
            jn = jnp.minimum(j + 2, _CPT - 1)
            refill(jn, dst_b, ew_b, rows_b, sem_b)
        return carry

    lax.fori_loop(0, _CPT // 2, pair, 0)
    drain(_CPT - 1, *bufs[0])
    drain(_CPT - 1, *bufs[1])
    plsc.subcore_barrier()
    for k in range(_RPT // _CHUNK):
        off = sid * _RPT + k * _CHUNK
        pltpu.sync_copy(acc_sh.at[pl.ds(off, _CHUNK)], rows0)
        pltpu.sync_copy(rows0, out_ref.at[cid, pl.ds(off, _CHUNK)])


_msg_call = pl.kernel(
    _msg_body,
    out_type=jax.ShapeDtypeStruct((2, _NPAD, _F), jnp.float32),
    mesh=plsc.VectorSubcoreMesh(core_axis_name="c", subcore_axis_name="s"),
    scratch_types=[
        pltpu.VMEM((_CPT, _CHUNK), jnp.int32),
        pltpu.VMEM((_CHUNK,), jnp.int32),
        pltpu.VMEM((_CHUNK,), jnp.int32),
        pltpu.VMEM((_CHUNK,), jnp.float32),
        pltpu.VMEM((_CHUNK,), jnp.float32),
        pltpu.VMEM((_CHUNK, _F), jnp.float32),
        pltpu.VMEM((_CHUNK, _F), jnp.float32),
        pltpu.SemaphoreType.DMA,
        pltpu.SemaphoreType.DMA,
        pltpu.VMEM_SHARED((_NPAD, _F), jnp.float32),
    ],
    compiler_params=pltpu.CompilerParams(needs_layout_passes=False),
)


def kernel(x, edge_index, edge_weight, p, W_ih, W_hh, b_ih, b_hh, W0, W_lin, b_lin):
    score = (x @ p) / jnp.linalg.norm(p)
    topv, perm = lax.top_k(score, _F)
    x_tilde = x[perm] * jnp.tanh(topv)[:, None]

    W = _gru_call(x_tilde, W_ih, W_hh, b_ih, b_hh, W0)

    pad = _EPAD - _E
    ncht = _NTILES * _CPT
    ss = jnp.concatenate(
        [edge_index[0], jnp.zeros((pad,), jnp.int32)]).reshape(ncht, _CHUNK)
    dd = jnp.concatenate(
        [edge_index[1], jnp.zeros((pad,), jnp.int32)]).reshape(ncht, _CHUNK)
    ew2 = jnp.concatenate(
        [edge_weight, jnp.zeros((pad,), jnp.float32)]).reshape(ncht, _CHUNK)

    degp = _deg_call(dd, ew2)
    xs, xd, dinv = _prep_call(x, W, degp)

    acc = _msg_call(ss, dd, ew2, xs)

    return _head_call(acc, xd, dinv, W_lin, b_lin)
